# baseline (speedup 1.0000x reference)
import functools

import jax
import jax.numpy as jnp
from jax import lax
from jax.experimental import pallas as pl
from jax.experimental.pallas import tpu as pltpu
from jax.experimental.pallas import tpu_sc as plsc

BATCH = 4096
SEQ = 50
DIM = 64
NUM_ROWS = BATCH * SEQ
NUM_WORKERS = 32
ROWS_PER_WORKER = NUM_ROWS // NUM_WORKERS
LANES = 128
CL = 5
NUM_CHUNKS = SEQ // CL
CROWS = CL * LANES
PITCH = 65


def _gather_kernel(idx_hbm, table_hbm, out_hbm, idx_v, glist_v, rows_v,
                   outb_v, gsem, wsem):
    wid = lax.axis_index("s") * 2 + lax.axis_index("c")
    rbase = wid * ROWS_PER_WORKER
    pltpu.sync_copy(idx_hbm.at[pl.ds(rbase, ROWS_PER_WORKER)], idx_v)

    lane_iota = lax.iota(jnp.int32, 16)

    def chunk_body(c, carry):
        l0 = c * CL
        for lp in range(CL):
            for i0 in range(0, LANES, 16):
                src = plsc.load_gather(
                    idx_v, [(i0 + lane_iota) * SEQ + (l0 + lp)])
                glist_v[pl.ds(lp * LANES + i0, 16)] = src
        pltpu.async_copy(table_hbm.at[glist_v], rows_v, gsem).wait()

        @plsc.parallel_loop(0, CL * 8, unroll=2)
        def _transpose(t):
            lp = t // 8
            i0 = (t % 8) * 16
            row_idx = lp * LANES + i0 + lane_iota
            for quarter in range(4):
                vs = []
                for k in range(16):
                    d = quarter * 16 + k
                    vs.append(plsc.load_gather(
                        rows_v, [row_idx, lane_iota * 0 + d]))
                for k in range(16):
                    d = quarter * 16 + k
                    outb_v[lp, d // 8, d % 8, pl.ds(i0, 16)] = vs[k]
        pltpu.async_copy(
            outb_v, out_hbm.at[pl.ds(l0, CL), slice(None), wid], wsem
        ).wait()
        return carry

    lax.fori_loop(0, NUM_CHUNKS, chunk_body, 0)


@jax.jit
def _lookup(idx_flat, table):
    mesh = plsc.VectorSubcoreMesh(core_axis_name="c", subcore_axis_name="s")
    run = functools.partial(
        pl.kernel,
        out_type=jax.ShapeDtypeStruct((SEQ, 8, NUM_WORKERS, 8, LANES),
                                      jnp.float32),
        mesh=mesh,
        scratch_types=[
            pltpu.VMEM((ROWS_PER_WORKER,), jnp.int32),
            pltpu.VMEM((CROWS,), jnp.int32),
            pltpu.VMEM((CROWS, PITCH), jnp.float32),
            pltpu.VMEM((CL, 8, 8, LANES), jnp.float32),
            pltpu.SemaphoreType.DMA,
            pltpu.SemaphoreType.DMA,
        ],
        compiler_params=pltpu.CompilerParams(use_tc_tiling_on_sc=False,
                                             needs_layout_passes=False,
                                             disable_bounds_checks=True),
    )(_gather_kernel)
    return run(idx_flat, table)


def kernel(indices, table):
    idx_flat = indices.reshape(-1).astype(jnp.int32)
    table_pad = jnp.pad(table, ((0, 0), (0, PITCH - DIM)))
    out5d = _lookup(idx_flat, table_pad)
    t = out5d.transpose(0, 1, 3, 2, 4)
    t = t.reshape(SEQ, DIM, BATCH)
    return t.transpose(2, 0, 1)


# speedup vs baseline: 2.0413x; 2.0413x over previous
import functools

import jax
import jax.numpy as jnp
from jax import lax
from jax.experimental import pallas as pl
from jax.experimental.pallas import tpu as pltpu
from jax.experimental.pallas import tpu_sc as plsc

BATCH = 4096
SEQ = 50
DIM = 64
NUM_ROWS = BATCH * SEQ
NUM_WORKERS = 32
ROWS_PER_WORKER = NUM_ROWS // NUM_WORKERS
LANES = 128
CL = 5
NUM_CHUNKS = SEQ // CL
CROWS = CL * LANES
OPITCH = 129


def _gather_kernel(idx_hbm, table_hbm, out_hbm, idx_v, glist_v, rows_v,
                   outb_v, gsem, wsem):
    wid = lax.axis_index("s") * 2 + lax.axis_index("c")
    rbase = wid * ROWS_PER_WORKER
    pltpu.sync_copy(idx_hbm.at[pl.ds(rbase, ROWS_PER_WORKER)], idx_v)

    lane_iota = lax.iota(jnp.int32, 16)

    def chunk_body(c, carry):
        l0 = c * CL
        for lp in range(CL):
            for i0 in range(0, LANES, 16):
                src = plsc.load_gather(
                    idx_v, [(i0 + lane_iota) * SEQ + (l0 + lp)])
                glist_v[pl.ds(lp * LANES + i0, 16)] = src
        pltpu.async_copy(table_hbm.at[glist_v], rows_v, gsem).wait()

        @plsc.parallel_loop(0, CROWS, unroll=4)
        def _scatter(t):
            lp = t >> 7
            i = t & (LANES - 1)
            for q in range(4):
                v = rows_v[t, pl.ds(16 * q, 16)]
                row = (16 * q + lane_iota) + lp * DIM
                col = lane_iota * 0 + i
                plsc.store_scatter(outb_v, [row, col], v)

        writes = []
        for lp in range(CL):
            for tr in range(8):
                writes.append(pltpu.async_copy(
                    outb_v.at[pl.ds(lp * DIM + tr * 8, 8), pl.ds(0, LANES)],
                    out_hbm.at[l0 + lp, tr, wid],
                    wsem))
        for w in writes:
            w.wait()
        return carry

    lax.fori_loop(0, NUM_CHUNKS, chunk_body, 0)


@jax.jit
def _lookup(idx_flat, table):
    mesh = plsc.VectorSubcoreMesh(core_axis_name="c", subcore_axis_name="s")
    run = functools.partial(
        pl.kernel,
        out_type=jax.ShapeDtypeStruct((SEQ, 8, NUM_WORKERS, 8, LANES),
                                      jnp.float32),
        mesh=mesh,
        scratch_types=[
            pltpu.VMEM((ROWS_PER_WORKER,), jnp.int32),
            pltpu.VMEM((CROWS,), jnp.int32),
            pltpu.VMEM((CROWS, DIM), jnp.float32),
            pltpu.VMEM((CL * DIM, OPITCH), jnp.float32),
            pltpu.SemaphoreType.DMA,
            pltpu.SemaphoreType.DMA,
        ],
        compiler_params=pltpu.CompilerParams(use_tc_tiling_on_sc=False,
                                             needs_layout_passes=False,
                                             disable_bounds_checks=True),
    )(_gather_kernel)
    return run(idx_flat, table)


def kernel(indices, table):
    idx_flat = indices.reshape(-1).astype(jnp.int32)
    out5d = _lookup(idx_flat, table)
    t = out5d.transpose(0, 1, 3, 2, 4)
    t = t.reshape(SEQ, DIM, BATCH)
    return t.transpose(2, 0, 1)


# speedup vs baseline: 2.2911x; 1.1224x over previous
import functools

import jax
import jax.numpy as jnp
from jax import lax
from jax.experimental import pallas as pl
from jax.experimental.pallas import tpu as pltpu
from jax.experimental.pallas import tpu_sc as plsc

BATCH = 4096
SEQ = 50
DIM = 64
NUM_ROWS = BATCH * SEQ
NUM_WORKERS = 32
ROWS_PER_WORKER = NUM_ROWS // NUM_WORKERS
LANES = 128
CL = 2
NUM_CHUNKS = SEQ // CL
CROWS = CL * LANES
OPITCH = 129


def _gather_kernel(idx_hbm, table_hbm, out_hbm, idx_v,
                   glist0, glist1, rows0, rows1, outb0, outb1,
                   gsem0, gsem1, wsem0, wsem1):
    wid = lax.axis_index("s") * 2 + lax.axis_index("c")
    rbase = wid * ROWS_PER_WORKER
    glist = (glist0, glist1)
    rows = (rows0, rows1)
    outb = (outb0, outb1)
    gsem = (gsem0, gsem1)
    wsem = (wsem0, wsem1)
    pltpu.sync_copy(idx_hbm.at[pl.ds(rbase, ROWS_PER_WORKER)], idx_v)

    lane_iota = lax.iota(jnp.int32, 16)

    def build_glist(p, chunk):
        l0 = chunk * CL
        for lp in range(CL):
            for i0 in range(0, LANES, 16):
                src = plsc.load_gather(
                    idx_v, [(i0 + lane_iota) * SEQ + (l0 + lp)])
                glist[p][pl.ds(lp * LANES + i0, 16)] = src

    def start_gather(p):
        return pltpu.async_copy(table_hbm.at[glist[p]], rows[p], gsem[p])

    def wait_gather(p):
        pltpu.make_async_copy(table_hbm.at[glist[p]], rows[p], gsem[p]).wait()

    def scatter_chunk(p):
        @plsc.parallel_loop(0, CROWS, unroll=4)
        def _scatter(t):
            lp = t >> 7
            i = t & (LANES - 1)
            for q in range(4):
                v = rows[p][t, pl.ds(16 * q, 16)]
                row = (16 * q + lane_iota) + lp * DIM
                col = lane_iota * 0 + i
                plsc.store_scatter(outb[p], [row, col], v)

    def issue_writes(p, l0):
        for lp in range(CL):
            for tr in range(8):
                pltpu.async_copy(
                    outb[p].at[pl.ds(lp * DIM + tr * 8, 8), pl.ds(0, LANES)],
                    out_hbm.at[l0 + lp, tr, wid],
                    wsem[p])

    def drain_writes(p, l0):
        for lp in range(CL):
            for tr in range(8):
                pltpu.make_async_copy(
                    outb[p].at[pl.ds(lp * DIM + tr * 8, 8), pl.ds(0, LANES)],
                    out_hbm.at[l0 + lp, tr, wid],
                    wsem[p]).wait()

    build_glist(0, 0)
    start_gather(0)

    def chunk_body(c, carry):
        for p in range(2):
            @pl.when((c & 1) == p)
            def _():
                wait_gather(p)
                @pl.when(c + 1 < NUM_CHUNKS)
                def _():
                    build_glist(1 - p, c + 1)
                    start_gather(1 - p)
                @pl.when(c >= 2)
                def _():
                    drain_writes(p, (c - 2) * CL)
                scatter_chunk(p)
                issue_writes(p, c * CL)
        return carry

    lax.fori_loop(0, NUM_CHUNKS, chunk_body, 0)
    drain_writes((NUM_CHUNKS - 2) & 1, (NUM_CHUNKS - 2) * CL)
    drain_writes((NUM_CHUNKS - 1) & 1, (NUM_CHUNKS - 1) * CL)


@jax.jit
def _lookup(idx_flat, table):
    mesh = plsc.VectorSubcoreMesh(core_axis_name="c", subcore_axis_name="s")
    run = functools.partial(
        pl.kernel,
        out_type=jax.ShapeDtypeStruct((SEQ, 8, NUM_WORKERS, 8, LANES),
                                      jnp.float32),
        mesh=mesh,
        scratch_types=[
            pltpu.VMEM((ROWS_PER_WORKER,), jnp.int32),
            pltpu.VMEM((CROWS,), jnp.int32),
            pltpu.VMEM((CROWS,), jnp.int32),
            pltpu.VMEM((CROWS, DIM), jnp.float32),
            pltpu.VMEM((CROWS, DIM), jnp.float32),
            pltpu.VMEM((CL * DIM, OPITCH), jnp.float32),
            pltpu.VMEM((CL * DIM, OPITCH), jnp.float32),
            pltpu.SemaphoreType.DMA,
            pltpu.SemaphoreType.DMA,
            pltpu.SemaphoreType.DMA,
            pltpu.SemaphoreType.DMA,
        ],
        compiler_params=pltpu.CompilerParams(use_tc_tiling_on_sc=False,
                                             needs_layout_passes=False,
                                             disable_bounds_checks=True),
    )(_gather_kernel)
    return run(idx_flat, table)


def kernel(indices, table):
    idx_flat = indices.reshape(-1).astype(jnp.int32)
    out5d = _lookup(idx_flat, table)
    t = out5d.transpose(0, 1, 3, 2, 4)
    t = t.reshape(SEQ, DIM, BATCH)
    return t.transpose(2, 0, 1)


# speedup vs baseline: 2.3153x; 1.0106x over previous
import functools

import jax
import jax.numpy as jnp
from jax import lax
from jax.experimental import pallas as pl
from jax.experimental.pallas import tpu as pltpu
from jax.experimental.pallas import tpu_sc as plsc

BATCH = 4096
SEQ = 50
DIM = 64
NUM_ROWS = BATCH * SEQ
NUM_WORKERS = 32
ROWS_PER_WORKER = NUM_ROWS // NUM_WORKERS
LANES = 128
CL = 2
NUM_CHUNKS = SEQ // CL
CROWS = CL * LANES
OPITCH = 129


def _gather_kernel(idx_hbm, table_hbm, out_hbm, idx_v,
                   rows0, rows1, outb0, outb1,
                   gsem0, gsem1, wsem0, wsem1):
    wid = lax.axis_index("s") * 2 + lax.axis_index("c")
    rows = (rows0, rows1)
    outb = (outb0, outb1)
    gsem = (gsem0, gsem1)
    wsem = (wsem0, wsem1)
    pltpu.sync_copy(idx_hbm.at[:, pl.ds(wid * LANES, LANES)], idx_v)

    lane_iota = lax.iota(jnp.int32, 16)

    def start_gather(p, chunk):
        l0 = chunk * CL
        for lp in range(CL):
            pltpu.async_copy(table_hbm.at[idx_v.at[l0 + lp]],
                             rows[p].at[pl.ds(lp * LANES, LANES)], gsem[p])

    def wait_gather(p):
        for lp in range(CL):
            pltpu.make_async_copy(table_hbm.at[idx_v.at[lp]],
                                  rows[p].at[pl.ds(lp * LANES, LANES)],
                                  gsem[p]).wait()

    def scatter_chunk(p):
        @plsc.parallel_loop(0, CROWS, unroll=4)
        def _scatter(t):
            lp = t >> 7
            i = t & (LANES - 1)
            for q in range(4):
                v = rows[p][t, pl.ds(16 * q, 16)]
                row = (16 * q + lane_iota) + lp * DIM
                col = lane_iota * 0 + i
                plsc.store_scatter(outb[p], [row, col], v)

    def issue_writes(p, l0):
        for lp in range(CL):
            for tr in range(8):
                pltpu.async_copy(
                    outb[p].at[pl.ds(lp * DIM + tr * 8, 8), pl.ds(0, LANES)],
                    out_hbm.at[l0 + lp, tr, wid],
                    wsem[p])

    def drain_writes(p, l0):
        for lp in range(CL):
            for tr in range(8):
                pltpu.make_async_copy(
                    outb[p].at[pl.ds(lp * DIM + tr * 8, 8), pl.ds(0, LANES)],
                    out_hbm.at[l0 + lp, tr, wid],
                    wsem[p]).wait()

    start_gather(0, 0)

    def chunk_body(c, carry):
        for p in range(2):
            @pl.when((c & 1) == p)
            def _():
                wait_gather(p)
                @pl.when(c + 1 < NUM_CHUNKS)
                def _():
                    start_gather(1 - p, c + 1)
                @pl.when(c >= 2)
                def _():
                    drain_writes(p, (c - 2) * CL)
                scatter_chunk(p)
                issue_writes(p, c * CL)
        return carry

    lax.fori_loop(0, NUM_CHUNKS, chunk_body, 0)
    drain_writes((NUM_CHUNKS - 2) & 1, (NUM_CHUNKS - 2) * CL)
    drain_writes((NUM_CHUNKS - 1) & 1, (NUM_CHUNKS - 1) * CL)


@jax.jit
def _lookup(idx_t, table):
    mesh = plsc.VectorSubcoreMesh(core_axis_name="c", subcore_axis_name="s")
    run = functools.partial(
        pl.kernel,
        out_type=jax.ShapeDtypeStruct((SEQ, 8, NUM_WORKERS, 8, LANES),
                                      jnp.float32),
        mesh=mesh,
        scratch_types=[
            pltpu.VMEM((SEQ, LANES), jnp.int32),
            pltpu.VMEM((CROWS, DIM), jnp.float32),
            pltpu.VMEM((CROWS, DIM), jnp.float32),
            pltpu.VMEM((CL * DIM, OPITCH), jnp.float32),
            pltpu.VMEM((CL * DIM, OPITCH), jnp.float32),
            pltpu.SemaphoreType.DMA,
            pltpu.SemaphoreType.DMA,
            pltpu.SemaphoreType.DMA,
            pltpu.SemaphoreType.DMA,
        ],
        compiler_params=pltpu.CompilerParams(use_tc_tiling_on_sc=False,
                                             needs_layout_passes=False,
                                             disable_bounds_checks=True),
    )(_gather_kernel)
    return run(idx_t, table)


def kernel(indices, table):
    idx_t = indices.T.astype(jnp.int32)
    out5d = _lookup(idx_t, table)
    t = out5d.transpose(0, 1, 3, 2, 4)
    t = t.reshape(SEQ, DIM, BATCH)
    return t.transpose(2, 0, 1)
